# baseline (speedup 1.0000x reference)
import functools

import jax
import jax.numpy as jnp
from jax import lax
from jax.experimental import pallas as pl
from jax.experimental.pallas import tpu as pltpu
from jax.experimental.pallas import tpu_sc as plsc

VOCAB = 100000
D = 64
B, V, C = 1024, 50, 20
NW = 32
BW = B // NW
G = C * BW
CV = C * V


def _body(x_hbm, table_hbm, out_hbm, idx_v,
          rows0, rows1, out0, out1, sem0, sem1):
    nc = 2
    wid = lax.axis_index("s") * nc + lax.axis_index("c")
    b0 = wid * BW
    rows_b = (rows0, rows1)
    out_b = (out0, out1)
    sem_b = (sem0, sem1)
    pltpu.sync_copy(x_hbm.at[:, pl.ds(b0, BW)], idx_v)

    def fire(v, buf):
        for c in range(C):
            pltpu.async_copy(
                table_hbm.at[idx_v.at[c * V + v]],
                rows_b[buf].at[pl.ds(c * BW, BW)], sem_b[buf])

    def drain(buf):
        pltpu.make_async_copy(
            table_hbm.at[pl.ds(0, G)], rows_b[buf], sem_b[buf]).wait()

    def step(v, buf):
        rows_v = rows_b[buf]
        out_v = out_b[buf]

        @pl.when(v + 1 < V)
        def _():
            fire(v + 1, 1 - buf)

        drain(buf)

        def acc_body(bl, _):
            for dw in range(D // 32):
                vals = [rows_v[c * BW + bl, pl.ds(dw * 32, 32)]
                        for c in range(C)]
                while len(vals) > 1:
                    vals = [vals[i] + vals[i + 1]
                            for i in range(0, len(vals) - 1, 2)]                         + ([vals[-1]] if len(vals) % 2 else [])
                out_v[bl, pl.ds(dw * 32, 32)] = vals[0]
            return 0

        lax.fori_loop(0, BW, acc_body, 0)
        pltpu.sync_copy(out_v, out_hbm.at[v, pl.ds(b0, BW), :])

    fire(0, 0)

    def outer(v0, _):
        for p in range(2):
            step(v0 + p, p)
        return 0

    lax.fori_loop(0, V // 2, lambda i, c: outer(i * 2, c), 0)


@jax.jit
def kernel(x, table):
    xq = x.astype(jnp.int32).transpose(2, 1, 0).reshape(CV, B)
    mesh = plsc.VectorSubcoreMesh(core_axis_name="c", subcore_axis_name="s")
    out = pl.kernel(
        _body,
        out_type=jax.ShapeDtypeStruct((V, B, D), jnp.bfloat16),
        mesh=mesh,
        compiler_params=pltpu.CompilerParams(use_tc_tiling_on_sc=False),
        scratch_types=[
            pltpu.VMEM((CV, BW), jnp.int32),
            pltpu.VMEM((G, 2 * D), jnp.bfloat16),
            pltpu.VMEM((G, 2 * D), jnp.bfloat16),
            pltpu.VMEM((BW, D), jnp.bfloat16),
            pltpu.VMEM((BW, D), jnp.bfloat16),
            pltpu.SemaphoreType.DMA,
            pltpu.SemaphoreType.DMA,
        ],
    )(xq, jnp.pad(table.astype(jnp.bfloat16), ((0, 7), (0, D))))
    return out.transpose(1, 0, 2).astype(jnp.float32)


# speedup vs baseline: 1.5722x; 1.5722x over previous
import functools

import jax
import jax.numpy as jnp
from jax import lax
from jax.experimental import pallas as pl
from jax.experimental.pallas import tpu as pltpu
from jax.experimental.pallas import tpu_sc as plsc

VOCAB = 100000
D = 64
B, V, C = 1024, 50, 20
NW = 32
BW = B // NW
G = C * BW
CV = C * V


def _body(x_hbm, table_hbm, out_hbm, idx_v,
          rows0, rows1, out0, out1, sem0, sem1):
    nc = 2
    wid = lax.axis_index("s") * nc + lax.axis_index("c")
    b0 = wid * BW
    rows_b = (rows0, rows1)
    out_b = (out0, out1)
    sem_b = (sem0, sem1)
    pltpu.sync_copy(x_hbm.at[:, pl.ds(b0, BW)], idx_v)

    def fire(v, buf):
        for c in range(C):
            pltpu.async_copy(
                table_hbm.at[idx_v.at[c * V + v]],
                rows_b[buf].at[pl.ds(c * BW, BW)], sem_b[buf])

    def drain(buf):
        pltpu.make_async_copy(
            table_hbm.at[pl.ds(0, G)], rows_b[buf], sem_b[buf]).wait()

    def step(v, buf):
        rows_v = rows_b[buf]
        out_v = out_b[buf]

        @pl.when(v + 1 < V)
        def _():
            fire(v + 1, 1 - buf)

        drain(buf)

        def acc_body(bl, _):
            for dw in range(D // 16):
                acc = rows_v[bl, pl.ds(dw * 16, 16)]
                for c in range(1, C):
                    acc = acc + rows_v[c * BW + bl, pl.ds(dw * 16, 16)]
                out_v[bl, pl.ds(dw * 16, 16)] = acc
            return 0

        lax.fori_loop(0, BW, acc_body, 0)
        pltpu.sync_copy(out_v, out_hbm.at[v, pl.ds(b0, BW), :])

    fire(0, 0)

    def outer(v0, _):
        for p in range(2):
            step(v0 + p, p)
        return 0

    lax.fori_loop(0, V // 2, lambda i, c: outer(i * 2, c), 0)


@jax.jit
def kernel(x, table):
    xq = x.astype(jnp.int32).transpose(2, 1, 0).reshape(CV, B)
    mesh = plsc.VectorSubcoreMesh(core_axis_name="c", subcore_axis_name="s")
    out = pl.kernel(
        _body,
        out_type=jax.ShapeDtypeStruct((V, B, D), jnp.float32),
        mesh=mesh,
        compiler_params=pltpu.CompilerParams(use_tc_tiling_on_sc=False),
        scratch_types=[
            pltpu.VMEM((CV, BW), jnp.int32),
            pltpu.VMEM((G, D), jnp.float32),
            pltpu.VMEM((G, D), jnp.float32),
            pltpu.VMEM((BW, D), jnp.float32),
            pltpu.VMEM((BW, D), jnp.float32),
            pltpu.SemaphoreType.DMA,
            pltpu.SemaphoreType.DMA,
        ],
    )(xq, table)
    return out.transpose(1, 0, 2)


# speedup vs baseline: 1.6371x; 1.0413x over previous
import functools

import jax
import jax.numpy as jnp
from jax import lax
from jax.experimental import pallas as pl
from jax.experimental.pallas import tpu as pltpu
from jax.experimental.pallas import tpu_sc as plsc

VOCAB = 100000
D = 64
B, V, C = 1024, 50, 20
NW = 32
BW = B // NW
G = C * BW
CV = C * V


def _body(x_hbm, table_hbm, out_hbm, idx_v,
          rows0, rows1, out0, out1, sem0, sem1):
    nc = 2
    wid = lax.axis_index("s") * nc + lax.axis_index("c")
    b0 = wid * BW
    rows_b = (rows0, rows1)
    out_b = (out0, out1)
    sem_b = (sem0, sem1)
    pltpu.sync_copy(x_hbm.at[:, pl.ds(b0, BW)], idx_v)

    def fire(v, buf):
        for c in range(C):
            pltpu.async_copy(
                table_hbm.at[idx_v.at[c * V + v]],
                rows_b[buf].at[pl.ds(c * BW, BW)], sem_b[buf])

    def drain(buf):
        pltpu.make_async_copy(
            table_hbm.at[pl.ds(0, G)], rows_b[buf], sem_b[buf]).wait()

    def step(v, buf):
        rows_v = rows_b[buf]
        out_v = out_b[buf]

        @pl.when(v + 1 < V)
        def _():
            fire(v + 1, 1 - buf)

        drain(buf)

        def acc_body(bl, _):
            for dw in range(D // 16):
                acc = rows_v[bl, pl.ds(dw * 16, 16)]
                for c in range(1, C):
                    acc = acc + rows_v[c * BW + bl, pl.ds(dw * 16, 16)]
                out_v[bl, pl.ds(dw * 16, 16)] = acc
            return 0

        lax.fori_loop(0, BW, acc_body, 0)
        pltpu.sync_copy(out_v, out_hbm.at[v, pl.ds(b0, BW), :])

    fire(0, 0)

    def outer(v0, _):
        for p in range(2):
            step(v0 + p, p)
        return 0

    lax.fori_loop(0, V // 2, lambda i, c: outer(i * 2, c), 0)


@jax.jit
def kernel(x, table):
    xq = (x.astype(jnp.int32) * 2).transpose(2, 1, 0).reshape(CV, B)
    mesh = plsc.VectorSubcoreMesh(core_axis_name="c", subcore_axis_name="s")
    out = pl.kernel(
        _body,
        out_type=jax.ShapeDtypeStruct((V, B, D), jnp.float32),
        mesh=mesh,
        compiler_params=pltpu.CompilerParams(use_tc_tiling_on_sc=False),
        scratch_types=[
            pltpu.VMEM((CV, BW), jnp.int32),
            pltpu.VMEM((G, D), jnp.float32),
            pltpu.VMEM((G, D), jnp.float32),
            pltpu.VMEM((BW, D), jnp.float32),
            pltpu.VMEM((BW, D), jnp.float32),
            pltpu.SemaphoreType.DMA,
            pltpu.SemaphoreType.DMA,
        ],
    )(xq, jnp.pad(table, ((0, 7), (0, D))).reshape(2 * (VOCAB + 8), D))
    return out.transpose(1, 0, 2)
